# Initial kernel scaffold; baseline (speedup 1.0000x reference)
import jax
import jax.numpy as jnp
from jax.experimental import pallas as pl


def kernel(link_state, states_graph_ids, states_first, states_second, sates_num_edges, Wm1, bm1, Wm2, bm2, gru_kernel, gru_rkernel, gru_bias, Wr1, br1, Wr2, br2, mask0):
    raise NotImplementedError("write your pallas kernel here")


# speedup vs baseline: 2.9805x; 2.9805x over previous
import functools

import jax
import jax.numpy as jnp
from jax import lax
from jax.experimental import pallas as pl
from jax.experimental.pallas import tpu as pltpu
from jax.experimental.pallas import tpu_sc as plsc

N = 10000
E = 320000
D = 128
G = 64
T = 4

NC = 2
NS = 16
NW = NC * NS
EPW = E // NW
K = 80
NCHUNK = EPW // K
NPAD = 10240
RPT = NPAD // NS

_mesh = plsc.VectorSubcoreMesh(core_axis_name="c", subcore_axis_name="s")


@functools.partial(
    pl.kernel,
    out_type=(
        jax.ShapeDtypeStruct((E, D), jnp.float32),
        jax.ShapeDtypeStruct((E, D), jnp.float32),
    ),
    mesh=_mesh,
    scratch_types=[
        pltpu.VMEM((K,), jnp.int32),
        pltpu.VMEM((K,), jnp.int32),
        pltpu.VMEM((K, D), jnp.float32),
        pltpu.VMEM((K, D), jnp.float32),
        pltpu.SemaphoreType.DMA,
        pltpu.SemaphoreType.DMA,
    ],
)
def _sc_gather(a_hbm, b_hbm, first_hbm, second_hbm, c1_hbm, c2_hbm,
               idx1, idx2, buf1, buf2, sem1, sem2):
    wid = lax.axis_index("s") * NC + lax.axis_index("c")

    def chunk(ci, _):
        base = (ci * NW + wid) * K
        pltpu.sync_copy(first_hbm.at[pl.ds(base, K)], idx1)
        pltpu.sync_copy(second_hbm.at[pl.ds(base, K)], idx2)
        cp1 = pltpu.async_copy(a_hbm.at[idx1], buf1, sem1)
        cp2 = pltpu.async_copy(b_hbm.at[idx2], buf2, sem2)
        cp1.wait()
        cp2.wait()
        pltpu.sync_copy(buf1, c1_hbm.at[pl.ds(base, K)])
        pltpu.sync_copy(buf2, c2_hbm.at[pl.ds(base, K)])
        return _

    lax.fori_loop(0, NCHUNK, chunk, None)


@functools.partial(
    pl.kernel,
    out_type=jax.ShapeDtypeStruct((NC, NPAD, D), jnp.float32),
    mesh=_mesh,
    scratch_types=[
        pltpu.VMEM((K,), jnp.int32),
        pltpu.VMEM((K, D), jnp.float32),
        pltpu.VMEM((RPT // 5, D), jnp.float32),
        pltpu.VMEM_SHARED((NPAD, D), jnp.float32),
    ],
)
def _sc_scatter(m_hbm, second_hbm, zero_hbm, out_hbm, idx, mbuf, stage, acc):
    cid = lax.axis_index("c")
    sid = lax.axis_index("s")
    wid = sid * NC + cid

    pltpu.sync_copy(zero_hbm, stage)
    for k in range(5):
        pltpu.sync_copy(stage, acc.at[pl.ds(sid * RPT + k * (RPT // 5), RPT // 5)])
    plsc.subcore_barrier()

    def chunk(ci, _):
        base = (ci * NW + wid) * K
        pltpu.sync_copy(second_hbm.at[pl.ds(base, K)], idx)
        pltpu.sync_copy(m_hbm.at[pl.ds(base, K)], mbuf)
        pltpu.sync_copy(mbuf, acc.at[idx], add=True)
        return _

    lax.fori_loop(0, NCHUNK, chunk, None)
    plsc.subcore_barrier()

    for k in range(5):
        r0 = sid * RPT + k * (RPT // 5)
        pltpu.sync_copy(acc.at[pl.ds(r0, RPT // 5)], stage)
        pltpu.sync_copy(stage, out_hbm.at[cid, pl.ds(r0, RPT // 5)])


BN = 2000
BE = 4000


def _proj_body(h_ref, w1a_ref, w1b_ref, b1_ref, a_ref, b_ref):
    h = h_ref[...]
    a_ref[...] = jnp.dot(h, w1a_ref[...], preferred_element_type=jnp.float32) + b1_ref[...]
    b_ref[...] = jnp.dot(h, w1b_ref[...], preferred_element_type=jnp.float32)


_proj = pl.pallas_call(
    _proj_body,
    grid=(N // BN,),
    in_specs=[
        pl.BlockSpec((BN, D), lambda i: (i, 0)),
        pl.BlockSpec((D, D), lambda i: (0, 0)),
        pl.BlockSpec((D, D), lambda i: (0, 0)),
        pl.BlockSpec((1, D), lambda i: (0, 0)),
    ],
    out_specs=[
        pl.BlockSpec((BN, D), lambda i: (i, 0)),
        pl.BlockSpec((BN, D), lambda i: (i, 0)),
    ],
    out_shape=[
        jax.ShapeDtypeStruct((N, D), jnp.float32),
        jax.ShapeDtypeStruct((N, D), jnp.float32),
    ],
)


def _edge_mlp_body(c1_ref, c2_ref, w2_ref, b2_ref, m_ref):
    c = jnp.maximum(c1_ref[...] + c2_ref[...], 0.0)
    m = jnp.dot(c, w2_ref[...], preferred_element_type=jnp.float32) + b2_ref[...]
    m_ref[...] = jnp.maximum(m, 0.0)


_edge_mlp = pl.pallas_call(
    _edge_mlp_body,
    grid=(E // BE,),
    in_specs=[
        pl.BlockSpec((BE, D), lambda i: (i, 0)),
        pl.BlockSpec((BE, D), lambda i: (i, 0)),
        pl.BlockSpec((D, D), lambda i: (0, 0)),
        pl.BlockSpec((1, D), lambda i: (0, 0)),
    ],
    out_specs=pl.BlockSpec((BE, D), lambda i: (i, 0)),
    out_shape=jax.ShapeDtypeStruct((E, D), jnp.float32),
)


def _gru_body(p_ref, h_ref, gk_ref, grk_ref, gb_ref, ho_ref):
    x = p_ref[0] + p_ref[1]
    h = h_ref[...]
    mx = jnp.dot(x, gk_ref[...], preferred_element_type=jnp.float32) + gb_ref[0:1, :]
    mh = jnp.dot(h, grk_ref[...], preferred_element_type=jnp.float32) + gb_ref[1:2, :]
    z = jax.nn.sigmoid(mx[:, :D] + mh[:, :D])
    r = jax.nn.sigmoid(mx[:, D:2 * D] + mh[:, D:2 * D])
    hh = jnp.tanh(mx[:, 2 * D:] + r * mh[:, 2 * D:])
    ho_ref[...] = z * h + (1.0 - z) * hh


_gru = pl.pallas_call(
    _gru_body,
    grid=(N // BN,),
    in_specs=[
        pl.BlockSpec((NC, BN, D), lambda i: (0, i, 0)),
        pl.BlockSpec((BN, D), lambda i: (i, 0)),
        pl.BlockSpec((D, 3 * D), lambda i: (0, 0)),
        pl.BlockSpec((D, 3 * D), lambda i: (0, 0)),
        pl.BlockSpec((2, 3 * D), lambda i: (0, 0)),
    ],
    out_specs=pl.BlockSpec((BN, D), lambda i: (i, 0)),
    out_shape=jax.ShapeDtypeStruct((N, D), jnp.float32),
)


def _readout_body(ids_ref, h_ref, wr1_ref, br1_ref, wr2_ref, br2_ref,
                  mask_ref, out_ref, acc_ref):
    i = pl.program_id(0)

    @pl.when(i == 0)
    def _zero():
        acc_ref[...] = jnp.zeros_like(acc_ref)

    ids = ids_ref[0]
    seg = lax.broadcasted_iota(jnp.int32, (G, 1), 0)
    onehot = (ids == seg).astype(jnp.float32)
    acc_ref[...] += jnp.dot(onehot, h_ref[...], preferred_element_type=jnp.float32)

    @pl.when(i == N // BN - 1)
    def _finish():
        t = jnp.dot(acc_ref[...], wr1_ref[...], preferred_element_type=jnp.float32)
        t = jnp.maximum(t + br1_ref[...], 0.0) * mask_ref[...]
        out_ref[...] = jnp.sum(t * wr2_ref[...], axis=1, keepdims=True) + br2_ref[...]


_readout = pl.pallas_call(
    _readout_body,
    grid=(N // BN,),
    in_specs=[
        pl.BlockSpec((1, 1, BN), lambda i: (i, 0, 0)),
        pl.BlockSpec((BN, D), lambda i: (i, 0)),
        pl.BlockSpec((D, D), lambda i: (0, 0)),
        pl.BlockSpec((1, D), lambda i: (0, 0)),
        pl.BlockSpec((1, D), lambda i: (0, 0)),
        pl.BlockSpec((1, 1), lambda i: (0, 0)),
        pl.BlockSpec((1, D), lambda i: (0, 0)),
    ],
    out_specs=pl.BlockSpec((G, 1), lambda i: (0, 0)),
    out_shape=jax.ShapeDtypeStruct((G, 1), jnp.float32),
    scratch_shapes=[pltpu.VMEM((G, D), jnp.float32)],
)


def kernel(link_state, states_graph_ids, states_first, states_second,
           sates_num_edges, Wm1, bm1, Wm2, bm2, gru_kernel, gru_rkernel,
           gru_bias, Wr1, br1, Wr2, br2, mask0):
    w1a = Wm1[:D]
    w1b = Wm1[D:]
    b1 = bm1.reshape(1, D)
    b2 = bm2.reshape(1, D)
    zeros_stage = jnp.zeros((RPT // 5, D), jnp.float32)

    h = link_state
    for _ in range(T):
        a, b = _proj(h, w1a, w1b, b1)
        c1, c2 = _sc_gather(a, b, states_first, states_second)
        m = _edge_mlp(c1, c2, Wm2, b2)
        p = _sc_scatter(m, states_second, zeros_stage)
        h = _gru(p, h, gru_kernel, gru_rkernel, gru_bias)

    out = _readout(states_graph_ids.reshape(N // BN, 1, BN), h, Wr1, br1.reshape(1, D),
                   Wr2.reshape(1, D), br2.reshape(1, 1), mask0.reshape(1, D))
    return out


# speedup vs baseline: 4.7496x; 1.5935x over previous
import functools

import jax
import jax.numpy as jnp
from jax import lax
from jax.experimental import pallas as pl
from jax.experimental.pallas import tpu as pltpu
from jax.experimental.pallas import tpu_sc as plsc

N = 10000
E = 320000
D = 128
G = 64
T = 4

NC = 2
NS = 16
NW = NC * NS
EPW = E // NW
K = 80
CPT = EPW // K
NPAD = 10240
RPT = NPAD // NS
SROWS = RPT // 5

_mesh = plsc.VectorSubcoreMesh(core_axis_name="c", subcore_axis_name="s")


@functools.partial(
    pl.kernel,
    out_type=(
        jax.ShapeDtypeStruct((E, D), jnp.float32),
        jax.ShapeDtypeStruct((E, D), jnp.float32),
    ),
    mesh=_mesh,
    scratch_types=[
        pltpu.VMEM((EPW,), jnp.int32),
        pltpu.VMEM((EPW,), jnp.int32),
        pltpu.VMEM((2, K, D), jnp.float32),
        pltpu.VMEM((2, K, D), jnp.float32),
        pltpu.SemaphoreType.DMA,
        pltpu.SemaphoreType.DMA,
    ],
)
def _sc_gather(a_hbm, b_hbm, first_hbm, second_hbm, c1_hbm, c2_hbm,
               idx_a, idx_b, buf_a, buf_b, sem_g, sem_w):
    wid = lax.axis_index("s") * NC + lax.axis_index("c")
    e0 = wid * EPW

    pltpu.sync_copy(first_hbm.at[pl.ds(e0, EPW)], idx_a)
    pltpu.sync_copy(second_hbm.at[pl.ds(e0, EPW)], idx_b)

    def start_gather(ci, slot):
        pltpu.async_copy(a_hbm.at[idx_a.at[pl.ds(ci * K, K)]], buf_a.at[slot], sem_g)
        pltpu.async_copy(b_hbm.at[idx_b.at[pl.ds(ci * K, K)]], buf_b.at[slot], sem_g)

    def wait_gather(slot):
        pltpu.make_async_copy(a_hbm.at[idx_a.at[pl.ds(0, K)]], buf_a.at[slot], sem_g).wait()
        pltpu.make_async_copy(b_hbm.at[idx_b.at[pl.ds(0, K)]], buf_b.at[slot], sem_g).wait()

    def start_write(ci, slot):
        base = e0 + ci * K
        pltpu.async_copy(buf_a.at[slot], c1_hbm.at[pl.ds(base, K)], sem_w)
        pltpu.async_copy(buf_b.at[slot], c2_hbm.at[pl.ds(base, K)], sem_w)

    def wait_write(slot):
        pltpu.make_async_copy(buf_a.at[slot], c1_hbm.at[pl.ds(0, K)], sem_w).wait()
        pltpu.make_async_copy(buf_b.at[slot], c2_hbm.at[pl.ds(0, K)], sem_w).wait()

    start_gather(0, 0)

    def body(ci, carry):
        slot = lax.rem(ci, 2)
        nslot = 1 - slot

        @pl.when(ci + 1 < CPT)
        def _():
            @pl.when(ci >= 1)
            def _():
                wait_write(nslot)
            start_gather(ci + 1, nslot)

        wait_gather(slot)
        start_write(ci, slot)
        return carry

    lax.fori_loop(0, CPT, body, None)
    wait_write(0)
    wait_write(1)


@functools.partial(
    pl.kernel,
    out_type=jax.ShapeDtypeStruct((NC, NPAD, D), jnp.float32),
    mesh=_mesh,
    scratch_types=[
        pltpu.VMEM((2, K), jnp.int32),
        pltpu.VMEM((2, K, D), jnp.float32),
        pltpu.VMEM((SROWS, D), jnp.float32),
        pltpu.VMEM_SHARED((NPAD, D), jnp.float32),
        pltpu.SemaphoreType.DMA,
    ],
)
def _sc_scatter(m_hbm, second_hbm, zero_hbm, out_hbm,
                idx_s, mbuf, stage, acc, sem_g):
    cid = lax.axis_index("c")
    sid = lax.axis_index("s")
    wid = sid * NC + cid
    e0 = wid * EPW

    pltpu.sync_copy(zero_hbm, stage)
    for k in range(5):
        pltpu.sync_copy(stage, acc.at[pl.ds(sid * RPT + k * SROWS, SROWS)])
    plsc.subcore_barrier()

    def start_load(ci, slot):
        base = e0 + ci * K
        pltpu.async_copy(second_hbm.at[pl.ds(base, K)], idx_s.at[slot], sem_g)
        pltpu.async_copy(m_hbm.at[pl.ds(base, K)], mbuf.at[slot], sem_g)

    def wait_load(slot):
        pltpu.make_async_copy(second_hbm.at[pl.ds(0, K)], idx_s.at[slot], sem_g).wait()
        pltpu.make_async_copy(m_hbm.at[pl.ds(0, K)], mbuf.at[slot], sem_g).wait()

    start_load(0, 0)

    def body(ci, carry):
        slot = lax.rem(ci, 2)

        @pl.when(ci + 1 < CPT)
        def _():
            start_load(ci + 1, 1 - slot)

        wait_load(slot)
        pltpu.sync_copy(mbuf.at[slot], acc.at[idx_s.at[slot]], add=True)
        return carry

    lax.fori_loop(0, CPT, body, None)
    plsc.subcore_barrier()

    for k in range(5):
        r0 = sid * RPT + k * SROWS
        pltpu.sync_copy(acc.at[pl.ds(r0, SROWS)], stage)
        pltpu.sync_copy(stage, out_hbm.at[cid, pl.ds(r0, SROWS)])


BN = 2000
BE = 4000


def _proj_body(h_ref, w1a_ref, w1b_ref, b1_ref, a_ref, b_ref):
    h = h_ref[...]
    a_ref[...] = jnp.dot(h, w1a_ref[...], preferred_element_type=jnp.float32) + b1_ref[...]
    b_ref[...] = jnp.dot(h, w1b_ref[...], preferred_element_type=jnp.float32)


_proj = pl.pallas_call(
    _proj_body,
    grid=(N // BN,),
    in_specs=[
        pl.BlockSpec((BN, D), lambda i: (i, 0)),
        pl.BlockSpec((D, D), lambda i: (0, 0)),
        pl.BlockSpec((D, D), lambda i: (0, 0)),
        pl.BlockSpec((1, D), lambda i: (0, 0)),
    ],
    out_specs=[
        pl.BlockSpec((BN, D), lambda i: (i, 0)),
        pl.BlockSpec((BN, D), lambda i: (i, 0)),
    ],
    out_shape=[
        jax.ShapeDtypeStruct((N, D), jnp.float32),
        jax.ShapeDtypeStruct((N, D), jnp.float32),
    ],
)


def _edge_mlp_body(c1_ref, c2_ref, w2_ref, b2_ref, m_ref):
    c = jnp.maximum(c1_ref[...] + c2_ref[...], 0.0)
    m = jnp.dot(c, w2_ref[...], preferred_element_type=jnp.float32) + b2_ref[...]
    m_ref[...] = jnp.maximum(m, 0.0)


_edge_mlp = pl.pallas_call(
    _edge_mlp_body,
    grid=(E // BE,),
    in_specs=[
        pl.BlockSpec((BE, D), lambda i: (i, 0)),
        pl.BlockSpec((BE, D), lambda i: (i, 0)),
        pl.BlockSpec((D, D), lambda i: (0, 0)),
        pl.BlockSpec((1, D), lambda i: (0, 0)),
    ],
    out_specs=pl.BlockSpec((BE, D), lambda i: (i, 0)),
    out_shape=jax.ShapeDtypeStruct((E, D), jnp.float32),
)


def _gru_proj_body(p_ref, h_ref, gk_ref, grk_ref, gb_ref,
                   w1a_ref, w1b_ref, b1_ref, ho_ref, a_ref, b_ref):
    x = p_ref[0] + p_ref[1]
    h = h_ref[...]
    mx = jnp.dot(x, gk_ref[...], preferred_element_type=jnp.float32) + gb_ref[0:1, :]
    mh = jnp.dot(h, grk_ref[...], preferred_element_type=jnp.float32) + gb_ref[1:2, :]
    z = jax.nn.sigmoid(mx[:, :D] + mh[:, :D])
    r = jax.nn.sigmoid(mx[:, D:2 * D] + mh[:, D:2 * D])
    hh = jnp.tanh(mx[:, 2 * D:] + r * mh[:, 2 * D:])
    hn = z * h + (1.0 - z) * hh
    ho_ref[...] = hn
    a_ref[...] = jnp.dot(hn, w1a_ref[...], preferred_element_type=jnp.float32) + b1_ref[...]
    b_ref[...] = jnp.dot(hn, w1b_ref[...], preferred_element_type=jnp.float32)


_gru_proj = pl.pallas_call(
    _gru_proj_body,
    grid=(N // BN,),
    in_specs=[
        pl.BlockSpec((NC, BN, D), lambda i: (0, i, 0)),
        pl.BlockSpec((BN, D), lambda i: (i, 0)),
        pl.BlockSpec((D, 3 * D), lambda i: (0, 0)),
        pl.BlockSpec((D, 3 * D), lambda i: (0, 0)),
        pl.BlockSpec((2, 3 * D), lambda i: (0, 0)),
        pl.BlockSpec((D, D), lambda i: (0, 0)),
        pl.BlockSpec((D, D), lambda i: (0, 0)),
        pl.BlockSpec((1, D), lambda i: (0, 0)),
    ],
    out_specs=[
        pl.BlockSpec((BN, D), lambda i: (i, 0)),
        pl.BlockSpec((BN, D), lambda i: (i, 0)),
        pl.BlockSpec((BN, D), lambda i: (i, 0)),
    ],
    out_shape=[
        jax.ShapeDtypeStruct((N, D), jnp.float32),
        jax.ShapeDtypeStruct((N, D), jnp.float32),
        jax.ShapeDtypeStruct((N, D), jnp.float32),
    ],
)


def _readout_body(ids_ref, h_ref, wr1_ref, br1_ref, wr2_ref, br2_ref,
                  mask_ref, out_ref, acc_ref):
    i = pl.program_id(0)

    @pl.when(i == 0)
    def _zero():
        acc_ref[...] = jnp.zeros_like(acc_ref)

    ids = ids_ref[0]
    seg = lax.broadcasted_iota(jnp.int32, (G, 1), 0)
    onehot = (ids == seg).astype(jnp.float32)
    acc_ref[...] += jnp.dot(onehot, h_ref[...], preferred_element_type=jnp.float32)

    @pl.when(i == N // BN - 1)
    def _finish():
        t = jnp.dot(acc_ref[...], wr1_ref[...], preferred_element_type=jnp.float32)
        t = jnp.maximum(t + br1_ref[...], 0.0) * mask_ref[...]
        out_ref[...] = jnp.sum(t * wr2_ref[...], axis=1, keepdims=True) + br2_ref[...]


_readout = pl.pallas_call(
    _readout_body,
    grid=(N // BN,),
    in_specs=[
        pl.BlockSpec((1, 1, BN), lambda i: (i, 0, 0)),
        pl.BlockSpec((BN, D), lambda i: (i, 0)),
        pl.BlockSpec((D, D), lambda i: (0, 0)),
        pl.BlockSpec((1, D), lambda i: (0, 0)),
        pl.BlockSpec((1, D), lambda i: (0, 0)),
        pl.BlockSpec((1, 1), lambda i: (0, 0)),
        pl.BlockSpec((1, D), lambda i: (0, 0)),
    ],
    out_specs=pl.BlockSpec((G, 1), lambda i: (0, 0)),
    out_shape=jax.ShapeDtypeStruct((G, 1), jnp.float32),
    scratch_shapes=[pltpu.VMEM((G, D), jnp.float32)],
)


def kernel(link_state, states_graph_ids, states_first, states_second,
           sates_num_edges, Wm1, bm1, Wm2, bm2, gru_kernel, gru_rkernel,
           gru_bias, Wr1, br1, Wr2, br2, mask0):
    w1a = Wm1[:D]
    w1b = Wm1[D:]
    b1 = bm1.reshape(1, D)
    b2 = bm2.reshape(1, D)
    zeros_stage = jnp.zeros((SROWS, D), jnp.float32)

    h = link_state
    a, b = _proj(h, w1a, w1b, b1)
    for _ in range(T):
        c1, c2 = _sc_gather(a, b, states_first, states_second)
        m = _edge_mlp(c1, c2, Wm2, b2)
        p = _sc_scatter(m, states_second, zeros_stage)
        h, a, b = _gru_proj(p, h, gru_kernel, gru_rkernel, gru_bias,
                            w1a, w1b, b1)

    out = _readout(states_graph_ids.reshape(N // BN, 1, BN), h, Wr1,
                   br1.reshape(1, D), Wr2.reshape(1, D), br2.reshape(1, 1),
                   mask0.reshape(1, D))
    return out
